# baseline (speedup 1.0000x reference)
import dataclasses
import functools

import jax
import jax.numpy as jnp
from jax import lax
from jax.experimental import pallas as pl
from jax.experimental.pallas import tpu as pltpu
from jax.experimental.pallas import tpu_sc as plsc

_N = 10000
_D = 128
_E = 320000
_NC = 2
_NS = 16
_NW = _NC * _NS
_EPW = _E // _NW
_K = 80
_NCHUNK = _EPW // _K
_NPAD = 10240
_RPT = _NPAD // _NS
_ZR = 16
_BN = 1000


def _sc_compiler_params():
    cp = pltpu.CompilerParams()
    if "needs_layout_passes" in pltpu.CompilerParams.__dataclass_fields__:
        cp = dataclasses.replace(cp, needs_layout_passes=False)
    return cp

@functools.cache
def _deg_kernel_fn():
    mesh = plsc.VectorSubcoreMesh(core_axis_name="c", subcore_axis_name="s",
                                  num_cores=_NC, num_subcores=_NS)
    return functools.partial(
        pl.kernel,
        out_type=jax.ShapeDtypeStruct((_NW, _NPAD), jnp.float32),
        mesh=mesh,
        scratch_types=[
            pltpu.VMEM((_EPW,), jnp.int32),
            pltpu.VMEM((_NPAD,), jnp.float32),
        ],
        compiler_params=_sc_compiler_params(),
    )(_deg_body)


def _deg_body(ei_hbm, out_hbm, dstv, hist):
    c = lax.axis_index("c")
    s = lax.axis_index("s")
    wid = s * _NC + c

    @pl.loop(0, _NPAD, step=16)
    def _(i):
        hist[pl.ds(i, 16)] = jnp.zeros((16,), jnp.float32)

    pltpu.sync_copy(ei_hbm.at[pl.ds(_E + wid * _EPW, _EPW)], dstv)
    ones = jnp.full((16,), 1.0, jnp.float32)

    @pl.loop(0, _EPW, step=16)
    def _(i):
        idx = dstv[pl.ds(i, 16)]
        plsc.addupdate_scatter(hist, [idx], ones)

    pltpu.sync_copy(hist, out_hbm.at[wid])


_BS = 1024


def _scale_kernel(degp):
    def body(p_ref, o_ref):
        ones = jnp.ones((_NW, 1), jnp.float32)
        deg = lax.dot_general(p_ref[...], ones, (((0,), (0,)), ((), ())),
                              preferred_element_type=jnp.float32)
        scale = 1.0 / jnp.maximum(deg, 1.0)
        o_ref[...] = jnp.broadcast_to(scale, (_BS, _D))

    return pl.pallas_call(
        body,
        grid=(_NPAD // _BS,),
        in_specs=[pl.BlockSpec((_NW, _BS), lambda i: (0, i))],
        out_specs=pl.BlockSpec((_BS, _D), lambda i: (i, 0)),
        out_shape=jax.ShapeDtypeStruct((_NPAD, _D), jnp.float32),
    )(degp)


_EH = _E // _NC
_RNG = _NPAD // _NS
_LCAP = 10608
_ACCR = 648
_WSZ = 2000
_NWIN = _EH // _WSZ
_GCH = 48
_NCH2 = _LCAP // _GCH


@functools.cache
def _filter_kernel_fn():
    mesh = plsc.VectorSubcoreMesh(core_axis_name="c", subcore_axis_name="s",
                                  num_cores=_NC, num_subcores=_NS)
    return functools.partial(
        pl.kernel,
        out_type=(jax.ShapeDtypeStruct((_NW, _LCAP), jnp.int32),
                  jax.ShapeDtypeStruct((_NW, _LCAP), jnp.int32)),
        mesh=mesh,
        scratch_types=[
            pltpu.VMEM((_WSZ,), jnp.int32),
            pltpu.VMEM((_WSZ,), jnp.int32),
            pltpu.VMEM((_WSZ,), jnp.int32),
            pltpu.VMEM((_WSZ,), jnp.int32),
            pltpu.VMEM((_LCAP,), jnp.int32),
            pltpu.VMEM((_LCAP,), jnp.int32),
            pltpu.SemaphoreType.DMA,
            pltpu.SemaphoreType.DMA,
        ],
        compiler_params=_sc_compiler_params(),
    )(_filter_body)


def _filter_body(ei_hbm, sl_hbm, dl_hbm,
                 sA, dA, sB, dB, slist, dlist, semA, semB):
    c = lax.axis_index("c")
    s = lax.axis_index("s")
    wid = c * _NS + s
    lo = s * _RNG
    ebase = c * _EH

    @pl.loop(0, _LCAP, step=16)
    def _(i):
        slist[pl.ds(i, 16)] = jnp.zeros((16,), jnp.int32)
        dlist[pl.ds(i, 16)] = jnp.full((16,), _RNG, jnp.int32)

    def fetch(w, sbuf, dbuf, sem):
        pltpu.async_copy(ei_hbm.at[pl.ds(ebase + w * _WSZ, _WSZ)], sbuf, sem)
        pltpu.async_copy(ei_hbm.at[pl.ds(_E + ebase + w * _WSZ, _WSZ)],
                         dbuf, sem)

    def wait_f(sbuf, dbuf, sem):
        pltpu.make_async_copy(ei_hbm.at[pl.ds(ebase, _WSZ)], sbuf, sem).wait()
        pltpu.make_async_copy(ei_hbm.at[pl.ds(ebase, _WSZ)], dbuf, sem).wait()

    def process(sbuf, dbuf, cnt0):
        def grp(g, cnt):
            srcv = sbuf[pl.ds(16 * g, 16)]
            dloc = dbuf[pl.ds(16 * g, 16)] - lo
            m = (dloc >= 0) & (dloc < _RNG)
            plsc.store_compressed(slist.at[pl.ds(cnt, 16)], srcv, mask=m)
            plsc.store_compressed(dlist.at[pl.ds(cnt, 16)], dloc, mask=m)
            cnt = cnt + jnp.sum(jnp.where(m, 1, 0))
            return jnp.minimum(cnt, _LCAP - 16)
        return pl.loop(0, _WSZ // 16, init_carry=cnt0)(grp)

    fetch(0, sA, dA, semA)

    def win(w2, cnt):
        w = 2 * w2
        fetch(w + 1, sB, dB, semB)
        wait_f(sA, dA, semA)
        cnt = process(sA, dA, cnt)

        @pl.when(w + 2 <= _NWIN - 1)
        def _():
            fetch(w + 2, sA, dA, semA)

        wait_f(sB, dB, semB)
        return process(sB, dB, cnt)

    pl.loop(0, _NWIN // 2, init_carry=jnp.int32(0))(win)

    pltpu.sync_copy(slist, sl_hbm.at[wid])
    pltpu.sync_copy(dlist, dl_hbm.at[wid])


@functools.cache
def _agg_kernel_fn():
    mesh = plsc.VectorSubcoreMesh(core_axis_name="c", subcore_axis_name="s",
                                  num_cores=_NC, num_subcores=_NS)
    return functools.partial(
        pl.kernel,
        out_type=jax.ShapeDtypeStruct((_NC, _NPAD, _D), jnp.float32),
        mesh=mesh,
        scratch_types=[
            pltpu.VMEM((_LCAP,), jnp.int32),
            pltpu.VMEM((_LCAP,), jnp.int32),
            pltpu.VMEM((_GCH, _D), jnp.float32),
            pltpu.VMEM((_GCH, _D), jnp.float32),
            pltpu.VMEM((_ACCR, _D), jnp.float32),
            pltpu.SemaphoreType.DMA,
            pltpu.SemaphoreType.DMA,
            pltpu.SemaphoreType.DMA,
        ],
        compiler_params=_sc_compiler_params(),
    )(_agg_body)


def _agg_body(h_hbm, sl_hbm, dl_hbm, out_hbm,
              slist, dlist, rb0, rb1, acc, sem0, sem1, semL):
    c = lax.axis_index("c")
    s = lax.axis_index("s")
    wid = c * _NS + s

    pltpu.async_copy(sl_hbm.at[wid], slist, semL)
    pltpu.async_copy(dl_hbm.at[wid], dlist, semL)

    zz = jnp.zeros((16,), jnp.float32)

    @pl.loop(0, _ACCR)
    def _(r):
        for j8 in range(_D // 16):
            acc[r, pl.ds(16 * j8, 16)] = zz

    pltpu.make_async_copy(sl_hbm.at[wid], slist, semL).wait()
    pltpu.make_async_copy(dl_hbm.at[wid], dlist, semL).wait()

    def gstart(ch, buf, sem):
        pltpu.async_copy(h_hbm.at[slist.at[pl.ds(ch * _GCH, _GCH)]],
                         buf, sem)

    def gwait(buf, sem):
        pltpu.make_async_copy(h_hbm.at[slist.at[pl.ds(0, _GCH)]],
                              buf, sem).wait()

    iota16 = lax.iota(jnp.int32, 16)

    def accum(ch, buf):
        @pl.loop(0, _GCH // 16)
        def _(g):
            dstv = dlist[pl.ds(ch * _GCH + 16 * g, 16)]
            rowv = iota16 + 16 * g

            def cbody(c2, colv):
                vals = plsc.load_gather(buf, [rowv, colv])
                plsc.addupdate_scatter(acc, [dstv, colv], vals)
                return colv + 1

            pl.loop(0, _D, init_carry=jnp.zeros((16,), jnp.int32),
                    unroll=8)(cbody)

    gstart(0, rb0, sem0)

    @pl.loop(0, (_NCH2 - 1) // 2)
    def _(t):
        ch0 = 2 * t
        gstart(ch0 + 1, rb1, sem1)
        gwait(rb0, sem0)
        accum(ch0, rb0)
        gstart(ch0 + 2, rb0, sem0)
        gwait(rb1, sem1)
        accum(ch0 + 1, rb1)

    gwait(rb0, sem0)
    accum(_NCH2 - 1, rb0)

    pltpu.sync_copy(acc.at[pl.ds(0, _RNG)],
                    out_hbm.at[c, pl.ds(s * _RNG, _RNG)])


def _tc_layer(aggp, scale2d, h, Wl, bl2, Wr, relu):
    def body(a_ref, sc_ref, h_ref, wl_ref, b_ref, wr_ref, o_ref):
        agg = (a_ref[0] + a_ref[1]) * sc_ref[...]
        acc = lax.dot_general(agg, wl_ref[...], (((1,), (1,)), ((), ())),
                              preferred_element_type=jnp.float32)
        acc = acc + lax.dot_general(h_ref[...], wr_ref[...],
                                    (((1,), (1,)), ((), ())),
                                    preferred_element_type=jnp.float32)
        acc = acc + b_ref[...]
        o_ref[...] = jnp.maximum(acc, 0.0) if relu else acc

    return pl.pallas_call(
        body,
        grid=(_N // _BN,),
        in_specs=[
            pl.BlockSpec((_NC, _BN, _D), lambda i: (0, i, 0)),
            pl.BlockSpec((_BN, _D), lambda i: (i, 0)),
            pl.BlockSpec((_BN, _D), lambda i: (i, 0)),
            pl.BlockSpec((_D, _D), lambda i: (0, 0)),
            pl.BlockSpec((1, _D), lambda i: (0, 0)),
            pl.BlockSpec((_D, _D), lambda i: (0, 0)),
        ],
        out_specs=pl.BlockSpec((_BN, _D), lambda i: (i, 0)),
        out_shape=jax.ShapeDtypeStruct((_N, _D), jnp.float32),
    )(aggp, scale2d, h, Wl, bl2, Wr)


def kernel(x, edge_index, Wl0, bl0, Wr0, Wl1, bl1, Wr1, Wl2, bl2, Wr2):
    ei = edge_index.astype(jnp.int32).reshape(2 * _E)

    degp = _deg_kernel_fn()(ei)
    scale2d = _scale_kernel(degp)
    sl, dl = _filter_kernel_fn()(ei)

    h = x
    for i, (Wl, bl, Wr) in enumerate(
            [(Wl0, bl0, Wr0), (Wl1, bl1, Wr1), (Wl2, bl2, Wr2)]):
        aggp = _agg_kernel_fn()(h, sl, dl)
        h = _tc_layer(aggp, scale2d, h, Wl, bl.reshape(1, _D), Wr,
                      relu=(i < 2))
    return h


# speedup vs baseline: 16.9324x; 16.9324x over previous
import dataclasses
import functools

import jax
import jax.numpy as jnp
from jax import lax
from jax.experimental import pallas as pl
from jax.experimental.pallas import tpu as pltpu
from jax.experimental.pallas import tpu_sc as plsc

_N = 10000
_D = 128
_E = 320000
_NC = 2
_NS = 16
_NW = _NC * _NS
_EPW = _E // _NW
_K = 80
_NCHUNK = _EPW // _K
_NPAD = 10240
_RPT = _NPAD // _NS
_ZR = 16
_BN = 2000


def _sc_compiler_params():
    cp = pltpu.CompilerParams()
    if "needs_layout_passes" in pltpu.CompilerParams.__dataclass_fields__:
        cp = dataclasses.replace(cp, needs_layout_passes=False)
    return cp

@functools.cache
def _deg_kernel_fn():
    mesh = plsc.VectorSubcoreMesh(core_axis_name="c", subcore_axis_name="s",
                                  num_cores=_NC, num_subcores=_NS)
    return functools.partial(
        pl.kernel,
        out_type=jax.ShapeDtypeStruct((_NW, _NPAD), jnp.float32),
        mesh=mesh,
        scratch_types=[
            pltpu.VMEM((_EPW,), jnp.int32),
            pltpu.VMEM((_NPAD,), jnp.float32),
        ],
        compiler_params=_sc_compiler_params(),
    )(_deg_body)


def _deg_body(ei_hbm, out_hbm, dstv, hist):
    c = lax.axis_index("c")
    s = lax.axis_index("s")
    wid = s * _NC + c

    @pl.loop(0, _NPAD, step=16)
    def _(i):
        hist[pl.ds(i, 16)] = jnp.zeros((16,), jnp.float32)

    pltpu.sync_copy(ei_hbm.at[pl.ds(_E + wid * _EPW, _EPW)], dstv)
    ones = jnp.full((16,), 1.0, jnp.float32)

    @pl.loop(0, _EPW, step=16)
    def _(i):
        idx = dstv[pl.ds(i, 16)]
        plsc.addupdate_scatter(hist, [idx], ones)

    pltpu.sync_copy(hist, out_hbm.at[wid])


_BS = 1024


def _scale_kernel(degp):
    def body(p_ref, o_ref):
        ones = jnp.ones((_NW, 1), jnp.float32)
        deg = lax.dot_general(p_ref[...], ones, (((0,), (0,)), ((), ())),
                              preferred_element_type=jnp.float32)
        scale = 1.0 / jnp.maximum(deg, 1.0)
        o_ref[...] = jnp.broadcast_to(scale, (_BS, _D))

    return pl.pallas_call(
        body,
        grid=(_NPAD // _BS,),
        in_specs=[pl.BlockSpec((_NW, _BS), lambda i: (0, i))],
        out_specs=pl.BlockSpec((_BS, _D), lambda i: (i, 0)),
        out_shape=jax.ShapeDtypeStruct((_NPAD, _D), jnp.float32),
    )(degp)


@functools.cache
def _agg_kernel_fn():
    mesh = plsc.VectorSubcoreMesh(core_axis_name="c", subcore_axis_name="s",
                                  num_cores=_NC, num_subcores=_NS)
    return functools.partial(
        pl.kernel,
        out_type=jax.ShapeDtypeStruct((_NC, _NPAD, _D), jnp.float32),
        mesh=mesh,
        scratch_types=[
            pltpu.VMEM((2, _K), jnp.int32),
            pltpu.VMEM((2, _K), jnp.int32),
            pltpu.VMEM((2, _K), jnp.int32),
            pltpu.VMEM((2, _K), jnp.int32),
            pltpu.VMEM((_K, _D), jnp.float32),
            pltpu.VMEM((_K, _D), jnp.float32),
            pltpu.VMEM_SHARED((_NPAD, _D), jnp.float32),
            pltpu.SemaphoreType.DMA,
            pltpu.SemaphoreType.DMA,
            pltpu.SemaphoreType.DMA,
            pltpu.SemaphoreType.DMA,
            pltpu.SemaphoreType.DMA,
            pltpu.SemaphoreType.DMA,
            pltpu.SemaphoreType.DMA,
            pltpu.SemaphoreType.DMA,
        ],
        compiler_params=_sc_compiler_params(),
    )(_agg_body)


def _agg_body(h_hbm, ei_hbm, z_hbm, out_hbm,
              i0, i1, i2, i3, rA, rB, acc,
              si0, si1, si2, si3, sRA, sRB, sSA, sSB):
    c = lax.axis_index("c")
    s = lax.axis_index("s")
    wid = s * _NC + c
    base = wid * _EPW

    pltpu.sync_copy(z_hbm.at[pl.ds(s * _RPT, _RPT)],
                    acc.at[pl.ds(s * _RPT, _RPT)])
    plsc.subcore_barrier()

    ibufs = (i0, i1, i2, i3)
    isems = (si0, si1, si2, si3)
    rbufs = (rA, rB)
    rsems = (sRA, sRB)

    def fetch_idx(buf, sem, chunk):
        pltpu.async_copy(ei_hbm.at[pl.ds(base + chunk * _K, _K)],
                         buf.at[0], sem)
        pltpu.async_copy(ei_hbm.at[pl.ds(_E + base + chunk * _K, _K)],
                         buf.at[1], sem)

    def wait_idx(buf, sem):
        pltpu.make_async_copy(ei_hbm.at[pl.ds(base, _K)],
                              buf.at[0], sem).wait()
        pltpu.make_async_copy(ei_hbm.at[pl.ds(base, _K)],
                              buf.at[1], sem).wait()

    ssems = (sSA, sSB)

    for b in range(4):
        fetch_idx(ibufs[b], isems[b], b)
    wait_idx(i0, si0)
    pltpu.async_copy(h_hbm.at[i0.at[0]], rA, sRA)

    @pl.loop(0, (_NCHUNK - 1) // 4)
    def _(t):
        for b in range(4):
            cch = 4 * t + b
            ib, si = ibufs[b], isems[b]
            rb, rs, ss = rbufs[b % 2], rsems[b % 2], ssems[b % 2]
            ib_n, si_n = ibufs[(b + 1) % 4], isems[(b + 1) % 4]
            ib_p, si_p = ibufs[(b + 3) % 4], isems[(b + 3) % 4]
            rb_n, rs_n = rbufs[(b + 1) % 2], rsems[(b + 1) % 2]
            ss_n = ssems[(b + 1) % 2]
            @pl.when(cch >= 1)
            def _():
                pltpu.make_async_copy(rb_n, acc.at[ib_p.at[1]],
                                      ss_n).wait()

            @pl.when((cch >= 1) & (cch + 3 <= _NCHUNK - 1))
            def _():
                fetch_idx(ib_p, si_p, cch + 3)
            wait_idx(ib_n, si_n)
            pltpu.async_copy(h_hbm.at[ib_n.at[0]], rb_n, rs_n)
            pltpu.make_async_copy(h_hbm.at[ib.at[0]], rb, rs).wait()
            pltpu.async_copy(rb, acc.at[ib.at[1]], ss, add=True)

    pltpu.make_async_copy(rB, acc.at[i3.at[1]], sSB).wait()
    pltpu.make_async_copy(h_hbm.at[i0.at[0]], rA, sRA).wait()
    pltpu.sync_copy(rA, acc.at[i0.at[1]], add=True)

    plsc.subcore_barrier()
    pltpu.sync_copy(acc.at[pl.ds(s * _RPT, _RPT)],
                    out_hbm.at[c, pl.ds(s * _RPT, _RPT)])


def _tc_layer(aggp, scale2d, h, Wl, bl2, Wr, relu):
    def body(a_ref, sc_ref, h_ref, wl_ref, b_ref, wr_ref, o_ref):
        agg = (a_ref[0] + a_ref[1]) * sc_ref[...]
        acc = lax.dot_general(agg, wl_ref[...], (((1,), (1,)), ((), ())),
                              preferred_element_type=jnp.float32)
        acc = acc + lax.dot_general(h_ref[...], wr_ref[...],
                                    (((1,), (1,)), ((), ())),
                                    preferred_element_type=jnp.float32)
        acc = acc + b_ref[...]
        o_ref[...] = jnp.maximum(acc, 0.0) if relu else acc

    return pl.pallas_call(
        body,
        grid=(_N // _BN,),
        in_specs=[
            pl.BlockSpec((_NC, _BN, _D), lambda i: (0, i, 0)),
            pl.BlockSpec((_BN, _D), lambda i: (i, 0)),
            pl.BlockSpec((_BN, _D), lambda i: (i, 0)),
            pl.BlockSpec((_D, _D), lambda i: (0, 0)),
            pl.BlockSpec((1, _D), lambda i: (0, 0)),
            pl.BlockSpec((_D, _D), lambda i: (0, 0)),
        ],
        out_specs=pl.BlockSpec((_BN, _D), lambda i: (i, 0)),
        out_shape=jax.ShapeDtypeStruct((_N, _D), jnp.float32),
    )(aggp, scale2d, h, Wl, bl2, Wr)


def kernel(x, edge_index, Wl0, bl0, Wr0, Wl1, bl1, Wr1, Wl2, bl2, Wr2):
    ei = edge_index.astype(jnp.int32).reshape(2 * _E)
    zeros = jnp.zeros((_NPAD, _D), jnp.float32)

    degp = _deg_kernel_fn()(ei)
    scale2d = _scale_kernel(degp)

    h = x
    for i, (Wl, bl, Wr) in enumerate(
            [(Wl0, bl0, Wr0), (Wl1, bl1, Wr1), (Wl2, bl2, Wr2)]):
        aggp = _agg_kernel_fn()(h, ei, zeros)
        h = _tc_layer(aggp, scale2d, h, Wl, bl.reshape(1, _D), Wr,
                      relu=(i < 2))
    return h
